# baseline (speedup 1.0000x reference)
import functools

import jax
import jax.numpy as jnp
from jax.experimental import pallas as pl

W = 16
EMB = 20
TOP_RATIO = 0.4
BLK = 512
bf16 = jnp.bfloat16
f32 = jnp.float32


def _endpoints(T, sentence_map, subtoken_map):
    starts = jnp.repeat(jnp.arange(T)[:, None], W, axis=1)
    ends = starts + jnp.arange(W)[None, :]
    start_sent = sentence_map[starts]
    end_sent = sentence_map[jnp.minimum(ends, T - 1)]
    c1 = ends < T
    c2 = start_sent == end_sent
    pad = -jnp.ones((W + 1,), dtype=subtoken_map.dtype)
    sub_pad = jnp.concatenate([subtoken_map, pad])
    c3 = sub_pad[starts] != sub_pad[starts - 1]
    c4 = sub_pad[ends] != sub_pad[ends + 1]
    mask = (c1 & c2 & c3 & c4).reshape(-1)
    return starts.reshape(-1), ends.reshape(-1), mask


def _mlp_kernel(se_ref, w1_ref, b1_ref, w2_ref, b2_ref, o_ref):
    se = se_ref[...].astype(bf16)
    w1 = w1_ref[...].astype(bf16)
    acc = jnp.dot(se, w1, preferred_element_type=f32)
    h = jnp.maximum(acc + b1_ref[0, :][None, :], 0.0)
    ment = jnp.dot(h.astype(bf16), w2_ref[...].astype(bf16),
                   preferred_element_type=f32) + b2_ref[0, 0]
    o_ref[...] = ment


def _pallas_mlp(se, w1, b1, w2, b2):
    n, k = se.shape
    d = w1.shape[1]
    w2p = jnp.pad(w2, ((0, 0), (0, 127)))
    out = pl.pallas_call(
        _mlp_kernel,
        grid=(n // BLK,),
        in_specs=[
            pl.BlockSpec((BLK, k), lambda i: (i, 0)),
            pl.BlockSpec((k, d), lambda i: (0, 0)),
            pl.BlockSpec((1, d), lambda i: (0, 0)),
            pl.BlockSpec((d, 128), lambda i: (0, 0)),
            pl.BlockSpec((1, 1), lambda i: (0, 0)),
        ],
        out_specs=pl.BlockSpec((BLK, 128), lambda i: (i, 0)),
        out_shape=jax.ShapeDtypeStruct((n, 128), f32),
    )(se, w1, b1.reshape(1, d), w2p, b2.reshape(1, 1))
    return out[:, 0]


def kernel(encoded_doc, sentence_map, subtoken_map, span_width_emb,
           span_width_prior_emb, attn_w, attn_b, m_w1, m_b1, m_w2, m_b2,
           w_w1, w_b1, w_w2, w_b2):
    T, d = encoded_doc.shape
    fs, fe, valid = _endpoints(T, sentence_map, subtoken_map)
    fe_safe = jnp.minimum(fe, T - 1)
    width_idx = jnp.minimum(fe - fs, W - 1)

    doc_range = jnp.arange(T)[None, :]
    ment_mask = (doc_range >= fs[:, None]) & (doc_range <= fe[:, None])
    word_attn = (encoded_doc @ attn_w + attn_b).squeeze(-1)
    attn_logits = (1.0 - ment_mask.astype(f32)) * -1e10 + word_attn[None, :]
    attn_probs = jax.nn.softmax(attn_logits, axis=1)
    attention_term = attn_probs @ encoded_doc

    span_embs = jnp.concatenate(
        [encoded_doc[fs], encoded_doc[fe_safe], span_width_emb[width_idx],
         attention_term], axis=1)
    ment_scores = _pallas_mlp(span_embs, m_w1, m_b1, m_w2, m_b2)

    prior_emb = span_width_prior_emb[width_idx]
    hw = jax.nn.relu(prior_emb @ w_w1 + w_b1)
    width_scores = (hw @ w_w2 + w_b2).squeeze(-1)

    total_scores = ment_scores + width_scores
    masked_scores = jnp.where(valid, total_scores, -jnp.inf)
    k = min(int(TOP_RATIO * T), int(masked_scores.shape[0]))
    topk_scores, topk_idx = jax.lax.top_k(masked_scores, k)
    return topk_scores, fs[topk_idx], fe[topk_idx]
